# baseline (speedup 1.0000x reference)
import jax
import jax.numpy as jnp
from jax import lax
from jax.experimental import pallas as pl
from jax.experimental.pallas import tpu as pltpu
from jax.experimental.pallas import tpu_sc as plsc

MAX_POS = 512
HEADS = 32
QLEN = 2048
KLEN = 2048
TBL = 2 * MAX_POS + 1
PAD = KLEN - MAX_POS - 1
W = 2112
NC, NS = 2, 16
NW = NC * NS
ROWS_PER_W = QLEN // NW
CHUNK = 128


def _body(table_hbm, out_hbm, extT, tstg, slab, tsem, sem):
    wid = lax.axis_index("s") * NC + lax.axis_index("c")
    q0 = wid * ROWS_PER_W
    j0 = (KLEN - ROWS_PER_W) - q0

    lanes = lax.iota(jnp.int32, 16)
    stride0 = lanes * W
    stride1 = (lanes + 16) * W

    row_lo = [None, None]
    row_hi = [None, None]
    for ch in range(0, TBL + 7, CHUNK):
        n = min(CHUNK, TBL + 7 - ch)
        pltpu.async_copy(table_hbm.at[pl.ds(ch, n), :], tstg.at[pl.ds(0, n), :],
                         tsem).wait()
        if ch == 0:
            row_lo = [tstg[0, pl.ds(0, 16)], tstg[0, pl.ds(16, 16)]]
        if ch == 1024:
            row_hi = [tstg[0, pl.ds(0, 16)], tstg[0, pl.ds(16, 16)]]

        def put_row(r, carry):
            v0 = tstg[r, pl.ds(0, 16)]
            v1 = tstg[r, pl.ds(16, 16)]
            pos = PAD + ch + r - j0
            ok = (pos >= 0) & (pos < W) & (lanes > -1)
            plsc.store_scatter(extT, [stride0 + pos], v0, mask=ok)
            plsc.store_scatter(extT, [stride1 + pos], v1, mask=ok)
            return carry

        lax.fori_loop(0, min(n, TBL - ch), put_row, 0)

    lo_end = jnp.clip(PAD - j0, 0, W)
    hi_start = jnp.clip(PAD + TBL - j0, 0, W)

    def put_lo(l, carry):
        plsc.store_scatter(extT, [stride0 + l], row_lo[0])
        plsc.store_scatter(extT, [stride1 + l], row_lo[1])
        return carry

    def put_hi(l, carry):
        plsc.store_scatter(extT, [stride0 + l], row_hi[0])
        plsc.store_scatter(extT, [stride1 + l], row_hi[1])
        return carry

    lax.fori_loop(0, lo_end, put_lo, 0)
    lax.fori_loop(hi_start, W, put_hi, 0)

    def do_q(r, carry):
        c = (ROWS_PER_W - 1) - r

        def build(hh):
            @plsc.parallel_loop(0, 16, unroll=8)
            def build_tile(kk):
                for hp in range(8):
                    base = (8 * hh + hp) * W + c + kk * 128
                    for l in range(8):
                        v = extT[pl.ds(base + l * 16, 16)]
                        slab[hh % 2, hp, pl.ds(kk * 128 + l * 16, 16)] = v

        def start(hh):
            cp = pltpu.make_async_copy(
                slab.at[hh % 2], out_hbm.at[q0 + r, pl.ds(8 * hh, 8), :], sem)
            cp.start()
            return cp

        build(0)
        cp0 = start(0)
        build(1)
        cp1 = start(1)
        cp0.wait()
        build(2)
        cp2 = start(2)
        cp1.wait()
        build(3)
        cp3 = start(3)
        cp2.wait()
        cp3.wait()
        return carry

    lax.fori_loop(0, ROWS_PER_W, do_q, 0)


@jax.jit
def _encode(table):
    mesh = plsc.VectorSubcoreMesh(core_axis_name="c", subcore_axis_name="s")
    run = pl.kernel(
        _body,
        mesh=mesh,
        out_type=jax.ShapeDtypeStruct((QLEN, HEADS, KLEN), jnp.float32),
        scratch_types=[
            pltpu.VMEM((HEADS * W,), jnp.float32),
            pltpu.VMEM((CHUNK, HEADS), jnp.float32),
            pltpu.VMEM((2, 8, KLEN), jnp.float32),
            pltpu.SemaphoreType.DMA,
            pltpu.SemaphoreType.DMA,
        ],
        compiler_params=pltpu.CompilerParams(needs_layout_passes=False),
    )
    return jnp.transpose(run(table), (0, 2, 1))


def kernel(seq_len_q, seq_len_k, embeddings_table):
    del seq_len_q, seq_len_k
    return _encode(embeddings_table)


# speedup vs baseline: 1.3701x; 1.3701x over previous
import jax
import jax.numpy as jnp
from jax import lax
from jax.experimental import pallas as pl
from jax.experimental.pallas import tpu as pltpu
from jax.experimental.pallas import tpu_sc as plsc

MAX_POS = 512
HEADS = 32
QLEN = 2048
KLEN = 2048
TBL = 2 * MAX_POS + 1
PAD = KLEN - MAX_POS - 1
W = 2112
NC, NS = 2, 16
NW = NC * NS
ROWS_PER_W = QLEN // NW
CHUNK = 128


def _body(table_hbm, out_hbm, extT, tstg, slab, tsem, sem):
    wid = lax.axis_index("s") * NC + lax.axis_index("c")
    q0 = wid * ROWS_PER_W
    j0 = (KLEN - ROWS_PER_W) - q0

    lanes = lax.iota(jnp.int32, 16)
    stride0 = lanes * W
    stride1 = (lanes + 16) * W

    row_lo = [None, None]
    row_hi = [None, None]
    for ch in range(0, TBL + 7, CHUNK):
        n = min(CHUNK, TBL + 7 - ch)
        pltpu.async_copy(table_hbm.at[pl.ds(ch, n), :], tstg.at[pl.ds(0, n), :],
                         tsem).wait()
        if ch == 0:
            row_lo = [tstg[0, pl.ds(0, 16)], tstg[0, pl.ds(16, 16)]]
        if ch == 1024:
            row_hi = [tstg[0, pl.ds(0, 16)], tstg[0, pl.ds(16, 16)]]

        def put_row(r, carry):
            v0 = tstg[r, pl.ds(0, 16)]
            v1 = tstg[r, pl.ds(16, 16)]
            pos = PAD + ch + r - j0
            ok = (pos >= 0) & (pos < W) & (lanes > -1)
            plsc.store_scatter(extT, [stride0 + pos], v0, mask=ok)
            plsc.store_scatter(extT, [stride1 + pos], v1, mask=ok)
            return carry

        lax.fori_loop(0, min(n, TBL - ch), put_row, 0)

    lo_end = jnp.clip(PAD - j0, 0, W)
    hi_start = jnp.clip(PAD + TBL - j0, 0, W)

    def put_lo(l, carry):
        plsc.store_scatter(extT, [stride0 + l], row_lo[0])
        plsc.store_scatter(extT, [stride1 + l], row_lo[1])
        return carry

    def put_hi(l, carry):
        plsc.store_scatter(extT, [stride0 + l], row_hi[0])
        plsc.store_scatter(extT, [stride1 + l], row_hi[1])
        return carry

    lax.fori_loop(0, lo_end, put_lo, 0)
    lax.fori_loop(hi_start, W, put_hi, 0)

    def do_q(r, carry):
        c = (ROWS_PER_W - 1) - r

        def build(hh):
            @plsc.parallel_loop(0, 16, unroll=4)
            def build_tile(kk):
                for hp in range(8):
                    base = (8 * hh + hp) * W + c + kk * 128
                    for l in range(8):
                        v = extT[pl.ds(base + l * 16, 16)]
                        slab[hh % 2, hp, pl.ds(kk * 128 + l * 16, 16)] = v

        def start(hh):
            cp = pltpu.make_async_copy(
                slab.at[hh % 2], out_hbm.at[q0 + r, pl.ds(8 * hh, 8), :], sem)
            cp.start()
            return cp

        build(0)
        cp0 = start(0)
        build(1)
        cp1 = start(1)
        cp0.wait()
        build(2)
        cp2 = start(2)
        cp1.wait()
        build(3)
        cp3 = start(3)
        cp2.wait()
        cp3.wait()
        return carry

    lax.fori_loop(0, ROWS_PER_W, do_q, 0)


@jax.jit
def _encode(table):
    mesh = plsc.VectorSubcoreMesh(core_axis_name="c", subcore_axis_name="s")
    run = pl.kernel(
        _body,
        mesh=mesh,
        out_type=jax.ShapeDtypeStruct((QLEN, HEADS, KLEN), jnp.float32),
        scratch_types=[
            pltpu.VMEM((HEADS * W,), jnp.float32),
            pltpu.VMEM((CHUNK, HEADS), jnp.float32),
            pltpu.VMEM((2, 8, KLEN), jnp.float32),
            pltpu.SemaphoreType.DMA,
            pltpu.SemaphoreType.DMA,
        ],
        compiler_params=pltpu.CompilerParams(needs_layout_passes=False),
    )
    return jnp.transpose(run(table), (0, 2, 1))


def kernel(seq_len_q, seq_len_k, embeddings_table):
    del seq_len_q, seq_len_k
    return _encode(embeddings_table)


# speedup vs baseline: 1.7496x; 1.2770x over previous
import jax
import jax.numpy as jnp
from jax import lax
from jax.experimental import pallas as pl
from jax.experimental.pallas import tpu as pltpu
from jax.experimental.pallas import tpu_sc as plsc

MAX_POS = 512
HEADS = 32
QLEN = 2048
KLEN = 2048
TBL = 2 * MAX_POS + 1
PAD = KLEN - MAX_POS - 1
W = 2112
NC, NS = 2, 16
NW = NC * NS
ROWS_PER_W = QLEN // NW
CHUNK = 128


def _body(table_hbm, out_hbm, extT, tstg, slab, tsem, sem):
    wid = lax.axis_index("s") * NC + lax.axis_index("c")
    q0 = wid * ROWS_PER_W
    j0 = (KLEN - ROWS_PER_W) - q0

    lanes = lax.iota(jnp.int32, 16)
    stride0 = lanes * W
    stride1 = (lanes + 16) * W

    row_lo = [None, None]
    row_hi = [None, None]
    for ch in range(0, TBL + 7, CHUNK):
        n = min(CHUNK, TBL + 7 - ch)
        pltpu.async_copy(table_hbm.at[pl.ds(ch, n), :], tstg.at[pl.ds(0, n), :],
                         tsem).wait()
        if ch == 0:
            row_lo = [tstg[0, pl.ds(0, 16)], tstg[0, pl.ds(16, 16)]]
        if ch == 1024:
            row_hi = [tstg[0, pl.ds(0, 16)], tstg[0, pl.ds(16, 16)]]

        def put_row(r, carry):
            v0 = tstg[r, pl.ds(0, 16)]
            v1 = tstg[r, pl.ds(16, 16)]
            pos = PAD + ch + r - j0
            ok = (pos >= 0) & (pos < W) & (lanes > -1)
            plsc.store_scatter(extT, [stride0 + pos], v0, mask=ok)
            plsc.store_scatter(extT, [stride1 + pos], v1, mask=ok)
            return carry

        lax.fori_loop(0, min(n, TBL - ch), put_row, 0)

    lo_end = jnp.clip(PAD - j0, 0, W)
    hi_start = jnp.clip(PAD + TBL - j0, 0, W)

    def put_lo(l, carry):
        plsc.store_scatter(extT, [stride0 + l], row_lo[0])
        plsc.store_scatter(extT, [stride1 + l], row_lo[1])
        return carry

    def put_hi(l, carry):
        plsc.store_scatter(extT, [stride0 + l], row_hi[0])
        plsc.store_scatter(extT, [stride1 + l], row_hi[1])
        return carry

    lax.fori_loop(0, lo_end, put_lo, 0)
    lax.fori_loop(hi_start, W, put_hi, 0)

    def do_q(r, carry):
        c = (ROWS_PER_W - 1) - r

        def build(hh):
            @plsc.parallel_loop(0, 128, unroll=8)
            def build_tile(t):
                kk = t // 8
                hp = t % 8
                base = (8 * hh + hp) * W + c + kk * 128
                for l in range(8):
                    v = extT[pl.ds(base + l * 16, 16)]
                    slab[hh % 2, hp, pl.ds(kk * 128 + l * 16, 16)] = v

        def start(hh):
            cp = pltpu.make_async_copy(
                slab.at[hh % 2], out_hbm.at[q0 + r, pl.ds(8 * hh, 8), :], sem)
            cp.start()
            return cp

        build(0)
        cp0 = start(0)
        build(1)
        cp1 = start(1)
        cp0.wait()
        build(2)
        cp2 = start(2)
        cp1.wait()
        build(3)
        cp3 = start(3)
        cp2.wait()
        cp3.wait()
        return carry

    lax.fori_loop(0, ROWS_PER_W, do_q, 0)


@jax.jit
def _encode(table):
    mesh = plsc.VectorSubcoreMesh(core_axis_name="c", subcore_axis_name="s")
    run = pl.kernel(
        _body,
        mesh=mesh,
        out_type=jax.ShapeDtypeStruct((QLEN, HEADS, KLEN), jnp.float32),
        scratch_types=[
            pltpu.VMEM((HEADS * W,), jnp.float32),
            pltpu.VMEM((CHUNK, HEADS), jnp.float32),
            pltpu.VMEM((2, 8, KLEN), jnp.float32),
            pltpu.SemaphoreType.DMA,
            pltpu.SemaphoreType.DMA,
        ],
        compiler_params=pltpu.CompilerParams(needs_layout_passes=False),
    )
    return jnp.transpose(run(table), (0, 2, 1))


def kernel(seq_len_q, seq_len_k, embeddings_table):
    del seq_len_q, seq_len_k
    return _encode(embeddings_table)
